# baseline (speedup 1.0000x reference)
import jax
import jax.numpy as jnp
from jax import lax
from jax.experimental import pallas as pl
from jax.experimental.pallas import tpu as pltpu
from jax.experimental.pallas import tpu_sc as plsc

B, D, C = 16384, 512, 100000
L = 16
NW = 32
CPW = C // NW
WREG = 3136
SEENP = WREG + L
NCHUNK = B // L
DCH = D // L
NBUF = 4


def _tc_copy(src_ref, dst_ref):
    dst_ref[...] = src_ref[...]


def _sc_prepare(x_hbm, y_hbm, mu_hbm, ck_hbm,
                rows_hbm, lbl_hbm, ckn_hbm, cnt_hbm,
                y_v, sel_v, wsel_v, seen_v, lbl_list, ckn_list,
                lbl_s, r_s, wcnt_s,
                mur0, mur1, mur2, mur3, xr0, xr1, xr2, xr3,
                ckg0, ckg1, ckg2, ckg3,
                gsem0, gsem1, gsem2, gsem3, wsem0, wsem1, wsem2, wsem3):
    buf = (
        (mur0, xr0, ckg0, gsem0, wsem0),
        (mur1, xr1, ckg1, gsem1, wsem1),
        (mur2, xr2, ckg2, gsem2, wsem2),
        (mur3, xr3, ckg3, gsem3, wsem3),
    )
    wid = lax.axis_index("s") * 2 + lax.axis_index("c")
    lo = wid * CPW
    hi = lo + CPW
    base = wid * WREG

    pltpu.sync_copy(y_hbm, y_v)

    lanes = lax.iota(jnp.int32, L)
    zeros = lanes * 0

    def zero_step(i, c2):
        seen_v[pl.ds(i * L, L)] = zeros
        return c2

    lax.fori_loop(0, SEENP // L, zero_step, jnp.int32(0))

    def sel_step(c, cnt):
        yv = y_v[pl.ds(c * L, L)]
        m = ((yv >= lo) & (yv < hi)).astype(jnp.int32)
        pos = jnp.where(m > 0, cnt + jnp.cumsum(m) - 1, B + L)
        plsc.store_scatter(sel_v, [pos], lanes + c * L)
        return cnt + jnp.sum(m)

    cnt = lax.fori_loop(0, NCHUNK, sel_step, jnp.int32(0))
    last = jnp.maximum(cnt - 1, 0)
    pad = plsc.load_gather(sel_v, [zeros + last])
    sel_v[pl.ds(cnt, L)] = pad
    ngroups = (cnt + (L - 1)) >> 4

    def win_step(gi, wcnt):
        g = ngroups - 1 - gi
        idx = sel_v[pl.ds(g * L, L)]
        lbl = plsc.load_gather(y_v, [idx])
        rel = lbl - lo
        lbl_s[...] = lbl
        seen = plsc.load_gather(seen_v, [rel])
        dup_later = lbl != lbl
        for s in range(1, L):
            perm = jnp.minimum(lanes + s, L - 1)
            rl = plsc.load_gather(lbl_s, [perm])
            dup_later = dup_later | ((rl == lbl) & (lanes < (L - s)))
        win = (~dup_later) & (seen == 0)
        plsc.store_scatter(seen_v, [rel], zeros + 1)
        wm = win.astype(jnp.int32)
        pos = jnp.where(win, wcnt + jnp.cumsum(wm) - 1, B + L)
        plsc.store_scatter(wsel_v, [pos], idx)
        return wcnt + jnp.sum(wm)

    wcnt = lax.fori_loop(0, ngroups, win_step, jnp.int32(0))
    wlast = jnp.maximum(wcnt - 1, 0)
    wpad = plsc.load_gather(wsel_v, [zeros + wlast])
    wsel_v[pl.ds(wcnt, L)] = wpad
    wgroups = (wcnt + (L - 1)) >> 4

    wcnt_s[...] = zeros + wgroups
    pltpu.sync_copy(wcnt_s, cnt_hbm.at[wid])

    def issue_gathers(g, k):
        mur, xr, ckg, gsem, _ = buf[k]
        idx = wsel_v[pl.ds(g * L, L)]
        lbl = plsc.load_gather(y_v, [idx])
        lbl_list[pl.ds(g * L, L)] = lbl
        pltpu.async_copy(mu_hbm.at[lbl], mur, gsem)
        pltpu.async_copy(x_hbm.at[idx], xr, gsem)
        pltpu.async_copy(ck_hbm.at[lbl], ckg, gsem)

    def process(g, k):
        mur, xr, ckg, gsem, wsem = buf[k]
        nk = (k + 1) % NBUF
        lbl = lbl_list[pl.ds(g * L, L)]
        pltpu.make_async_copy(mu_hbm.at[lbl], mur, gsem).wait()
        pltpu.make_async_copy(x_hbm.at[lbl], xr, gsem).wait()
        pltpu.make_async_copy(ck_hbm.at[lbl], ckg, gsem).wait()

        @pl.when(g + 1 < wgroups)
        def _():
            @pl.when(g >= NBUF - 1)
            def _():
                pmur, _, _, _, pwsem = buf[nk]
                pltpu.make_async_copy(
                    pmur, rows_hbm.at[pl.ds(base, L)], pwsem).wait()

            issue_gathers(g + 1, nk)

        ck1 = ckg[...] + 1.0
        r_s[...] = 1.0 / ck1
        ckn_list[pl.ds(g * L, L)] = ck1

        def row_step(j, c2):
            rj = plsc.load_gather(r_s, [zeros + j])
            for cpos in range(DCH):
                mu = mur[j, pl.ds(cpos * L, L)]
                xx = xr[j, pl.ds(cpos * L, L)]
                mur[j, pl.ds(cpos * L, L)] = mu + (xx - mu) * rj
            return c2

        lax.fori_loop(0, L, row_step, jnp.int32(0))

        pltpu.async_copy(mur, rows_hbm.at[pl.ds(base + g * L, L)], wsem)

    @pl.when(wgroups > 0)
    def _():
        issue_gathers(0, 0)

    def quad_step(p, carry):
        for k in range(NBUF):
            g = p * NBUF + k

            @pl.when(g < wgroups)
            def _(g=g, k=k):
                process(g, k)

        return carry

    lax.fori_loop(0, (wgroups + (NBUF - 1)) // NBUF, quad_step, jnp.int32(0))

    for t in range(1, NBUF + 1):
        for k in range(NBUF):
            @pl.when((wgroups >= t) & ((wgroups - t) % NBUF == k))
            def _(k=k):
                mur, _, _, _, wsem = buf[k]
                pltpu.make_async_copy(
                    mur, rows_hbm.at[pl.ds(base, L)], wsem).wait()

    pltpu.sync_copy(lbl_list, lbl_hbm.at[pl.ds(base, WREG)])
    pltpu.sync_copy(ckn_list, ckn_hbm.at[pl.ds(base, WREG)])


def _sc_scatter(rows_hbm, lbl_hbm, ckn_hbm, cnt_hbm, mu_out, ck_out,
                lbl_list, ckn_list, wcnt_s,
                rb0, rb1, rb2, rb3,
                gsem0, gsem1, gsem2, gsem3, ssem0, ssem1, ssem2, ssem3):
    buf = ((rb0, gsem0, ssem0), (rb1, gsem1, ssem1),
           (rb2, gsem2, ssem2), (rb3, gsem3, ssem3))
    wid = lax.axis_index("s") * 2 + lax.axis_index("c")
    base = wid * WREG

    pltpu.sync_copy(cnt_hbm.at[wid], wcnt_s)
    pltpu.sync_copy(lbl_hbm.at[pl.ds(base, WREG)], lbl_list)
    pltpu.sync_copy(ckn_hbm.at[pl.ds(base, WREG)], ckn_list)
    wgroups = jnp.max(wcnt_s[...])

    def issue_gather(g, k):
        rb, gsem, _ = buf[k]
        pltpu.async_copy(rows_hbm.at[pl.ds(base + g * L, L)], rb, gsem)

    def process(g, k):
        rb, gsem, _ = buf[k]
        nk = (k + 1) % NBUF
        pltpu.make_async_copy(rows_hbm.at[pl.ds(base, L)], rb, gsem).wait()

        @pl.when(g + 1 < wgroups)
        def _():
            @pl.when(g >= NBUF - 1)
            def _():
                prb, _, pssem = buf[nk]
                lp = lbl_list[pl.ds((g + 1 - NBUF) * L, L)]
                pltpu.make_async_copy(prb, mu_out.at[lp], pssem).wait()
                pltpu.make_async_copy(
                    ckn_list.at[pl.ds((g + 1 - NBUF) * L, L)],
                    ck_out.at[lp], pssem).wait()

            issue_gather(g + 1, nk)

        lbl = lbl_list[pl.ds(g * L, L)]
        ssem = buf[k][2]
        pltpu.async_copy(rb, mu_out.at[lbl], ssem)
        pltpu.async_copy(ckn_list.at[pl.ds(g * L, L)], ck_out.at[lbl], ssem)

    @pl.when(wgroups > 0)
    def _():
        issue_gather(0, 0)

    def quad_step(p, carry):
        for k in range(NBUF):
            g = p * NBUF + k

            @pl.when(g < wgroups)
            def _(g=g, k=k):
                process(g, k)

        return carry

    lax.fori_loop(0, (wgroups + (NBUF - 1)) // NBUF, quad_step, jnp.int32(0))

    for t in range(1, NBUF + 1):
        for k in range(NBUF):
            @pl.when((wgroups >= t) & ((wgroups - t) % NBUF == k))
            def _(t=t, k=k):
                rb, _, ssem = buf[k]
                lp = lbl_list[pl.ds((wgroups - t) * L, L)]
                pltpu.make_async_copy(rb, mu_out.at[lp], ssem).wait()
                pltpu.make_async_copy(
                    ckn_list.at[pl.ds((wgroups - t) * L, L)],
                    ck_out.at[lp], ssem).wait()


def kernel(x, y, muK, cK):
    mesh = plsc.VectorSubcoreMesh(core_axis_name="c", subcore_axis_name="s",
                                  num_cores=2, num_subcores=16)
    cp = pltpu.CompilerParams(needs_layout_passes=False)
    vec16i = pltpu.VMEM((L,), jnp.int32)
    vec16f = pltpu.VMEM((L,), jnp.float32)
    rows = pltpu.VMEM((L, D), jnp.float32)
    dma = pltpu.SemaphoreType.DMA

    mu_copy = pl.pallas_call(
        _tc_copy,
        grid=(100,),
        in_specs=[pl.BlockSpec((1000, D), lambda i: (i, 0))],
        out_specs=pl.BlockSpec((1000, D), lambda i: (i, 0)),
        out_shape=jax.ShapeDtypeStruct((C, D), jnp.float32),
    )(muK)

    rows_b, lbl_b, ckn_b, cnt_b = pl.kernel(
        _sc_prepare,
        out_type=(
            jax.ShapeDtypeStruct((NW * WREG, D), jnp.float32),
            jax.ShapeDtypeStruct((NW * WREG,), jnp.int32),
            jax.ShapeDtypeStruct((NW * WREG,), jnp.float32),
            jax.ShapeDtypeStruct((NW, L), jnp.int32),
        ),
        mesh=mesh,
        compiler_params=cp,
        scratch_types=[
            pltpu.VMEM((B,), jnp.int32),
            pltpu.VMEM((B + 2 * L,), jnp.int32),
            pltpu.VMEM((B + 2 * L,), jnp.int32),
            pltpu.VMEM((SEENP,), jnp.int32),
            pltpu.VMEM((WREG,), jnp.int32),
            pltpu.VMEM((WREG,), jnp.float32),
            vec16i, vec16f, vec16i,
            rows, rows, rows, rows,
            rows, rows, rows, rows,
            vec16f, vec16f, vec16f, vec16f,
            dma, dma, dma, dma, dma, dma, dma, dma,
        ],
    )(x, y, muK, cK)

    mu_out = jax.new_ref(mu_copy)
    ck_out = jax.new_ref(cK)
    pl.kernel(
        _sc_scatter,
        out_type=(),
        mesh=mesh,
        compiler_params=cp,
        scratch_types=[
            pltpu.VMEM((WREG,), jnp.int32),
            pltpu.VMEM((WREG,), jnp.float32),
            vec16i,
            rows, rows, rows, rows,
            dma, dma, dma, dma, dma, dma, dma, dma,
        ],
    )(rows_b, lbl_b, ckn_b, cnt_b, mu_out, ck_out)
    return mu_out[...], ck_out[...]


# speedup vs baseline: 1.0035x; 1.0035x over previous
import jax
import jax.numpy as jnp
from jax import lax
from jax.experimental import pallas as pl
from jax.experimental.pallas import tpu as pltpu
from jax.experimental.pallas import tpu_sc as plsc

B, D, C = 16384, 512, 100000
L = 16
NW = 32
CPW = C // NW
WREG = 3136
SEENP = WREG + L
NCHUNK = B // L
DCH = D // L
NBUF = 4
BAT = 64
GPB = BAT // L
SNBUF = 3


def _tc_copy(src_ref, dst_ref):
    dst_ref[...] = src_ref[...]


def _sc_prepare(x_hbm, y_hbm, mu_hbm, ck_hbm,
                rows_hbm, lbl_hbm, ckn_hbm, cnt_hbm,
                y_v, sel_v, wsel_v, seen_v, lbl_list, ckn_list,
                lbl_s, r_s, wcnt_s,
                mur0, mur1, mur2, mur3, xr0, xr1, xr2, xr3,
                ckg0, ckg1, ckg2, ckg3,
                gsem0, gsem1, gsem2, gsem3, wsem0, wsem1, wsem2, wsem3):
    buf = (
        (mur0, xr0, ckg0, gsem0, wsem0),
        (mur1, xr1, ckg1, gsem1, wsem1),
        (mur2, xr2, ckg2, gsem2, wsem2),
        (mur3, xr3, ckg3, gsem3, wsem3),
    )
    wid = lax.axis_index("s") * 2 + lax.axis_index("c")
    lo = wid * CPW
    hi = lo + CPW
    base = wid * WREG

    pltpu.sync_copy(y_hbm, y_v)

    lanes = lax.iota(jnp.int32, L)
    zeros = lanes * 0

    def zero_step(i, c2):
        seen_v[pl.ds(i * L, L)] = zeros
        return c2

    lax.fori_loop(0, SEENP // L, zero_step, jnp.int32(0))

    def sel_step(c, cnt):
        yv = y_v[pl.ds(c * L, L)]
        m = ((yv >= lo) & (yv < hi)).astype(jnp.int32)
        pos = jnp.where(m > 0, cnt + jnp.cumsum(m) - 1, B + L)
        plsc.store_scatter(sel_v, [pos], lanes + c * L)
        return cnt + jnp.sum(m)

    cnt = lax.fori_loop(0, NCHUNK, sel_step, jnp.int32(0))
    last = jnp.maximum(cnt - 1, 0)
    pad = plsc.load_gather(sel_v, [zeros + last])
    sel_v[pl.ds(cnt, L)] = pad
    ngroups = (cnt + (L - 1)) >> 4

    def win_step(gi, wcnt):
        g = ngroups - 1 - gi
        idx = sel_v[pl.ds(g * L, L)]
        lbl = plsc.load_gather(y_v, [idx])
        rel = lbl - lo
        lbl_s[...] = lbl
        seen = plsc.load_gather(seen_v, [rel])
        dup_later = lbl != lbl
        for s in range(1, L):
            perm = jnp.minimum(lanes + s, L - 1)
            rl = plsc.load_gather(lbl_s, [perm])
            dup_later = dup_later | ((rl == lbl) & (lanes < (L - s)))
        win = (~dup_later) & (seen == 0)
        plsc.store_scatter(seen_v, [rel], zeros + 1)
        wm = win.astype(jnp.int32)
        pos = jnp.where(win, wcnt + jnp.cumsum(wm) - 1, B + L)
        plsc.store_scatter(wsel_v, [pos], idx)
        return wcnt + jnp.sum(wm)

    wcnt = lax.fori_loop(0, ngroups, win_step, jnp.int32(0))
    wlast = jnp.maximum(wcnt - 1, 0)
    wpad = plsc.load_gather(wsel_v, [zeros + wlast])
    wsel_v[pl.ds(wcnt, L)] = wpad
    wgroups = (wcnt + (L - 1)) >> 4

    wcnt_s[...] = zeros + wgroups
    pltpu.sync_copy(wcnt_s, cnt_hbm.at[wid])

    def issue_gathers(g, k):
        mur, xr, ckg, gsem, _ = buf[k]
        idx = wsel_v[pl.ds(g * L, L)]
        lbl = plsc.load_gather(y_v, [idx])
        lbl_list[pl.ds(g * L, L)] = lbl
        pltpu.async_copy(mu_hbm.at[lbl], mur, gsem)
        pltpu.async_copy(x_hbm.at[idx], xr, gsem)
        pltpu.async_copy(ck_hbm.at[lbl], ckg, gsem)

    def process(g, k):
        mur, xr, ckg, gsem, wsem = buf[k]
        nk = (k + 1) % NBUF
        lbl = lbl_list[pl.ds(g * L, L)]
        pltpu.make_async_copy(mu_hbm.at[lbl], mur, gsem).wait()
        pltpu.make_async_copy(x_hbm.at[lbl], xr, gsem).wait()
        pltpu.make_async_copy(ck_hbm.at[lbl], ckg, gsem).wait()

        @pl.when(g + 1 < wgroups)
        def _():
            @pl.when(g >= NBUF - 1)
            def _():
                pmur, _, _, _, pwsem = buf[nk]
                pltpu.make_async_copy(
                    pmur, rows_hbm.at[pl.ds(base, L)], pwsem).wait()

            issue_gathers(g + 1, nk)

        ck1 = ckg[...] + 1.0
        r_s[...] = 1.0 / ck1
        ckn_list[pl.ds(g * L, L)] = ck1

        def row_step(j, c2):
            rj = plsc.load_gather(r_s, [zeros + j])
            for cpos in range(DCH):
                mu = mur[j, pl.ds(cpos * L, L)]
                xx = xr[j, pl.ds(cpos * L, L)]
                mur[j, pl.ds(cpos * L, L)] = mu + (xx - mu) * rj
            return c2

        lax.fori_loop(0, L, row_step, jnp.int32(0))

        pltpu.async_copy(mur, rows_hbm.at[pl.ds(base + g * L, L)], wsem)

    @pl.when(wgroups > 0)
    def _():
        issue_gathers(0, 0)

    def quad_step(p, carry):
        for k in range(NBUF):
            g = p * NBUF + k

            @pl.when(g < wgroups)
            def _(g=g, k=k):
                process(g, k)

        return carry

    lax.fori_loop(0, (wgroups + (NBUF - 1)) // NBUF, quad_step, jnp.int32(0))

    for t in range(1, NBUF + 1):
        for k in range(NBUF):
            @pl.when((wgroups >= t) & ((wgroups - t) % NBUF == k))
            def _(k=k):
                mur, _, _, _, wsem = buf[k]
                pltpu.make_async_copy(
                    mur, rows_hbm.at[pl.ds(base, L)], wsem).wait()

    ngpad = ((wgroups + (GPB - 1)) >> 2) << 2
    for k in range(NBUF):
        @pl.when((wgroups > 0) & ((wgroups - 1) % NBUF == k))
        def _(k=k):
            mur = buf[k][0]
            wsem = buf[k][4]
            lseg = lbl_list[pl.ds((wgroups - 1) * L, L)]
            cseg = ckn_list[pl.ds((wgroups - 1) * L, L)]

            def pad_step(g, c2):
                lbl_list[pl.ds(g * L, L)] = lseg
                ckn_list[pl.ds(g * L, L)] = cseg
                pltpu.sync_copy(mur, rows_hbm.at[pl.ds(base + g * L, L)])
                return c2

            lax.fori_loop(wgroups, ngpad, pad_step, jnp.int32(0))

    pltpu.sync_copy(lbl_list, lbl_hbm.at[pl.ds(base, WREG)])
    pltpu.sync_copy(ckn_list, ckn_hbm.at[pl.ds(base, WREG)])


def _sc_scatter(rows_hbm, lbl_hbm, ckn_hbm, cnt_hbm, mu_out, ck_out,
                lbl_list, ckn_list, wcnt_s,
                rb0, rb1, rb2, ib0, ib1, ib2,
                gsem0, gsem1, gsem2, ssem0, ssem1, ssem2):
    buf = ((rb0, ib0, gsem0, ssem0), (rb1, ib1, gsem1, ssem1),
           (rb2, ib2, gsem2, ssem2))
    wid = lax.axis_index("s") * 2 + lax.axis_index("c")
    base = wid * WREG

    pltpu.sync_copy(cnt_hbm.at[wid], wcnt_s)
    pltpu.sync_copy(lbl_hbm.at[pl.ds(base, WREG)], lbl_list)
    pltpu.sync_copy(ckn_hbm.at[pl.ds(base, WREG)], ckn_list)
    wgroups = jnp.max(wcnt_s[...])
    nbat = (wgroups + (GPB - 1)) >> 2

    def issue_gather(b, k):
        rb, _, gsem, _ = buf[k]
        pltpu.async_copy(rows_hbm.at[pl.ds(base + b * BAT, BAT)], rb, gsem)

    def drain_scatter(k):
        rb, ib, _, ssem = buf[k]
        pltpu.make_async_copy(rb, mu_out.at[ib], ssem).wait()
        pltpu.make_async_copy(
            ckn_list.at[pl.ds(0, BAT)], ck_out.at[ib], ssem).wait()

    def process(b, k):
        rb, ib, gsem, ssem = buf[k]
        nk = (k + 1) % SNBUF
        pltpu.make_async_copy(
            rows_hbm.at[pl.ds(base, BAT)], rb, gsem).wait()

        @pl.when(b + 1 < nbat)
        def _():
            @pl.when(b >= SNBUF - 1)
            def _():
                drain_scatter(nk)

            issue_gather(b + 1, nk)

        for q in range(GPB):
            ib[pl.ds(q * L, L)] = lbl_list[pl.ds(b * BAT + q * L, L)]
        pltpu.async_copy(rb, mu_out.at[ib], ssem)
        pltpu.async_copy(ckn_list.at[pl.ds(b * BAT, BAT)],
                         ck_out.at[ib], ssem)

    @pl.when(nbat > 0)
    def _():
        issue_gather(0, 0)

    def tri_step(p, carry):
        for k in range(SNBUF):
            b = p * SNBUF + k

            @pl.when(b < nbat)
            def _(b=b, k=k):
                process(b, k)

        return carry

    lax.fori_loop(0, (nbat + (SNBUF - 1)) // SNBUF, tri_step, jnp.int32(0))

    for t in range(1, SNBUF + 1):
        for k in range(SNBUF):
            @pl.when((nbat >= t) & ((nbat - t) % SNBUF == k))
            def _(k=k):
                drain_scatter(k)


def kernel(x, y, muK, cK):
    mesh = plsc.VectorSubcoreMesh(core_axis_name="c", subcore_axis_name="s",
                                  num_cores=2, num_subcores=16)
    cp = pltpu.CompilerParams(needs_layout_passes=False)
    vec16i = pltpu.VMEM((L,), jnp.int32)
    vec16f = pltpu.VMEM((L,), jnp.float32)
    rows = pltpu.VMEM((L, D), jnp.float32)
    dma = pltpu.SemaphoreType.DMA

    mu_copy = pl.pallas_call(
        _tc_copy,
        grid=(50,),
        in_specs=[pl.BlockSpec((2000, D), lambda i: (i, 0))],
        out_specs=pl.BlockSpec((2000, D), lambda i: (i, 0)),
        out_shape=jax.ShapeDtypeStruct((C, D), jnp.float32),
    )(muK)

    rows_b, lbl_b, ckn_b, cnt_b = pl.kernel(
        _sc_prepare,
        out_type=(
            jax.ShapeDtypeStruct((NW * WREG, D), jnp.float32),
            jax.ShapeDtypeStruct((NW * WREG,), jnp.int32),
            jax.ShapeDtypeStruct((NW * WREG,), jnp.float32),
            jax.ShapeDtypeStruct((NW, L), jnp.int32),
        ),
        mesh=mesh,
        compiler_params=cp,
        scratch_types=[
            pltpu.VMEM((B,), jnp.int32),
            pltpu.VMEM((B + 2 * L,), jnp.int32),
            pltpu.VMEM((B + 2 * L,), jnp.int32),
            pltpu.VMEM((SEENP,), jnp.int32),
            pltpu.VMEM((WREG,), jnp.int32),
            pltpu.VMEM((WREG,), jnp.float32),
            vec16i, vec16f, vec16i,
            rows, rows, rows, rows,
            rows, rows, rows, rows,
            vec16f, vec16f, vec16f, vec16f,
            dma, dma, dma, dma, dma, dma, dma, dma,
        ],
    )(x, y, muK, cK)

    mu_out = jax.new_ref(mu_copy)
    ck_out = jax.new_ref(cK)
    brows = pltpu.VMEM((BAT, D), jnp.float32)
    bidx = pltpu.VMEM((BAT,), jnp.int32)
    pl.kernel(
        _sc_scatter,
        out_type=(),
        mesh=mesh,
        compiler_params=cp,
        scratch_types=[
            pltpu.VMEM((WREG,), jnp.int32),
            pltpu.VMEM((WREG,), jnp.float32),
            vec16i,
            brows, brows, brows,
            bidx, bidx, bidx,
            dma, dma, dma, dma, dma, dma,
        ],
    )(rows_b, lbl_b, ckn_b, cnt_b, mu_out, ck_out)
    return mu_out[...], ck_out[...]


# speedup vs baseline: 1.0717x; 1.0680x over previous
import jax
import jax.numpy as jnp
from jax import lax
from jax.experimental import pallas as pl
from jax.experimental.pallas import tpu as pltpu
from jax.experimental.pallas import tpu_sc as plsc

B, D, C = 16384, 512, 100000
L = 16
NW = 32
CPW = C // NW
WREG = 3136
SEENP = WREG + L
NCHUNK = B // L
DCH = D // L
NBUF = 4


def _tc_copy(src_ref, dst_ref):
    dst_ref[...] = src_ref[...]


def _sc_select(y_hbm, idx_hbm, lbl_hbm, cnt_hbm,
               y_v, sel_v, seen_v, idx_list, lbl_list, lbl_s, wcnt_s):
    wid = lax.axis_index("s") * 2 + lax.axis_index("c")
    lo = wid * CPW
    hi = lo + CPW
    base = wid * WREG

    pltpu.sync_copy(y_hbm, y_v)

    lanes = lax.iota(jnp.int32, L)
    zeros = lanes * 0

    def zero_step(i, c2):
        seen_v[pl.ds(i * L, L)] = zeros
        return c2

    lax.fori_loop(0, SEENP // L, zero_step, jnp.int32(0))

    def sel_step(c, cnt):
        yv = y_v[pl.ds(c * L, L)]
        m = ((yv >= lo) & (yv < hi)).astype(jnp.int32)
        pos = jnp.where(m > 0, cnt + jnp.cumsum(m) - 1, B + L)
        plsc.store_scatter(sel_v, [pos], lanes + c * L)
        return cnt + jnp.sum(m)

    cnt = lax.fori_loop(0, NCHUNK, sel_step, jnp.int32(0))
    last = jnp.maximum(cnt - 1, 0)
    pad = plsc.load_gather(sel_v, [zeros + last])
    sel_v[pl.ds(cnt, L)] = pad
    ngroups = (cnt + (L - 1)) >> 4

    def win_step(gi, wcnt):
        g = ngroups - 1 - gi
        idx = sel_v[pl.ds(g * L, L)]
        lbl = plsc.load_gather(y_v, [idx])
        rel = lbl - lo
        lbl_s[...] = lbl
        seen = plsc.load_gather(seen_v, [rel])
        dup_later = lbl != lbl
        for s in range(1, L):
            perm = jnp.minimum(lanes + s, L - 1)
            rl = plsc.load_gather(lbl_s, [perm])
            dup_later = dup_later | ((rl == lbl) & (lanes < (L - s)))
        win = (~dup_later) & (seen == 0)
        plsc.store_scatter(seen_v, [rel], zeros + 1)
        wm = win.astype(jnp.int32)
        wpos = jnp.where(win, wcnt + jnp.cumsum(wm) - 1, WREG + L)
        plsc.store_scatter(idx_list, [wpos], idx)
        plsc.store_scatter(lbl_list, [wpos], lbl)
        return wcnt + jnp.sum(wm)

    wcnt = lax.fori_loop(0, ngroups, win_step, jnp.int32(0))

    wlast = jnp.maximum(wcnt - 1, 0)
    ipad = plsc.load_gather(idx_list, [zeros + wlast])
    lpad = plsc.load_gather(lbl_list, [zeros + wlast])
    idx_list[pl.ds(wcnt, L)] = ipad
    lbl_list[pl.ds(wcnt, L)] = lpad
    wgroups = (wcnt + (L - 1)) >> 4

    wcnt_s[...] = zeros + wgroups
    pltpu.sync_copy(wcnt_s, cnt_hbm.at[wid])
    pltpu.sync_copy(idx_list.at[pl.ds(0, WREG)], idx_hbm.at[pl.ds(base, WREG)])
    pltpu.sync_copy(lbl_list.at[pl.ds(0, WREG)], lbl_hbm.at[pl.ds(base, WREG)])


def _sc_update(x_hbm, mu_hbm, ck_hbm, idx_hbm, lbl_hbm, cnt_hbm,
               mu_out, ck_out,
               idx_list, lbl_list, r_s, wcnt_s,
               mur0, mur1, mur2, mur3, xr0, xr1, xr2, xr3,
               ckg0, ckg1, ckg2, ckg3, ckn0, ckn1, ckn2, ckn3,
               gsem0, gsem1, gsem2, gsem3, ssem0, ssem1, ssem2, ssem3):
    buf = (
        (mur0, xr0, ckg0, ckn0, gsem0, ssem0),
        (mur1, xr1, ckg1, ckn1, gsem1, ssem1),
        (mur2, xr2, ckg2, ckn2, gsem2, ssem2),
        (mur3, xr3, ckg3, ckn3, gsem3, ssem3),
    )
    wid = lax.axis_index("s") * 2 + lax.axis_index("c")
    base = wid * WREG

    pltpu.sync_copy(cnt_hbm.at[wid], wcnt_s)
    pltpu.sync_copy(idx_hbm.at[pl.ds(base, WREG)], idx_list)
    pltpu.sync_copy(lbl_hbm.at[pl.ds(base, WREG)], lbl_list)
    wgroups = jnp.max(wcnt_s[...])

    lanes = lax.iota(jnp.int32, L)
    zeros = lanes * 0

    def issue_gathers(g, k):
        mur, xr, ckg, _, gsem, _ = buf[k]
        idx = idx_list[pl.ds(g * L, L)]
        lbl = lbl_list[pl.ds(g * L, L)]
        pltpu.async_copy(mu_hbm.at[lbl], mur, gsem)
        pltpu.async_copy(x_hbm.at[idx], xr, gsem)
        pltpu.async_copy(ck_hbm.at[lbl], ckg, gsem)

    def process(g, k):
        mur, xr, ckg, ckn, gsem, ssem = buf[k]
        nk = (k + 1) % NBUF
        lbl = lbl_list[pl.ds(g * L, L)]
        pltpu.make_async_copy(mu_hbm.at[lbl], mur, gsem).wait()
        pltpu.make_async_copy(x_hbm.at[lbl], xr, gsem).wait()
        pltpu.make_async_copy(ck_hbm.at[lbl], ckg, gsem).wait()

        @pl.when(g + 1 < wgroups)
        def _():
            @pl.when(g >= NBUF - 1)
            def _():
                pmur, _, _, pckn, _, pssem = buf[nk]
                pltpu.make_async_copy(pmur, mu_out.at[lbl], pssem).wait()
                pltpu.make_async_copy(pckn, ck_out.at[lbl], pssem).wait()

            issue_gathers(g + 1, nk)

        ck1 = ckg[...] + 1.0
        r_s[...] = 1.0 / ck1
        ckn[...] = ck1

        def row_step(j, c2):
            rj = plsc.load_gather(r_s, [zeros + j])
            for cpos in range(DCH):
                mu = mur[j, pl.ds(cpos * L, L)]
                xx = xr[j, pl.ds(cpos * L, L)]
                mur[j, pl.ds(cpos * L, L)] = mu + (xx - mu) * rj
            return c2

        lax.fori_loop(0, L, row_step, jnp.int32(0))

        pltpu.async_copy(mur, mu_out.at[lbl], ssem)
        pltpu.async_copy(ckn, ck_out.at[lbl], ssem)

    @pl.when(wgroups > 0)
    def _():
        issue_gathers(0, 0)

    def quad_step(p, carry):
        for k in range(NBUF):
            g = p * NBUF + k

            @pl.when(g < wgroups)
            def _(g=g, k=k):
                process(g, k)

        return carry

    lax.fori_loop(0, (wgroups + (NBUF - 1)) // NBUF, quad_step, jnp.int32(0))

    dummy = lbl_list[pl.ds(0, L)]
    for t in range(1, NBUF + 1):
        for k in range(NBUF):
            @pl.when((wgroups >= t) & ((wgroups - t) % NBUF == k))
            def _(k=k):
                mur, _, _, ckn, _, ssem = buf[k]
                pltpu.make_async_copy(mur, mu_out.at[dummy], ssem).wait()
                pltpu.make_async_copy(ckn, ck_out.at[dummy], ssem).wait()


def kernel(x, y, muK, cK):
    mesh = plsc.VectorSubcoreMesh(core_axis_name="c", subcore_axis_name="s",
                                  num_cores=2, num_subcores=16)
    cp = pltpu.CompilerParams(needs_layout_passes=False)
    vec16i = pltpu.VMEM((L,), jnp.int32)
    vec16f = pltpu.VMEM((L,), jnp.float32)
    rows = pltpu.VMEM((L, D), jnp.float32)
    dma = pltpu.SemaphoreType.DMA

    mu_copy = pl.pallas_call(
        _tc_copy,
        grid=(100,),
        in_specs=[pl.BlockSpec((1000, D), lambda i: (i, 0))],
        out_specs=pl.BlockSpec((1000, D), lambda i: (i, 0)),
        out_shape=jax.ShapeDtypeStruct((C, D), jnp.float32),
    )(muK)

    idx_b, lbl_b, cnt_b = pl.kernel(
        _sc_select,
        out_type=(
            jax.ShapeDtypeStruct((NW * WREG,), jnp.int32),
            jax.ShapeDtypeStruct((NW * WREG,), jnp.int32),
            jax.ShapeDtypeStruct((NW, L), jnp.int32),
        ),
        mesh=mesh,
        compiler_params=cp,
        scratch_types=[
            pltpu.VMEM((B,), jnp.int32),
            pltpu.VMEM((B + 2 * L,), jnp.int32),
            pltpu.VMEM((SEENP,), jnp.int32),
            pltpu.VMEM((WREG + 2 * L,), jnp.int32),
            pltpu.VMEM((WREG + 2 * L,), jnp.int32),
            vec16i, vec16i,
        ],
    )(y)

    mu_out = jax.new_ref(mu_copy)
    ck_out = jax.new_ref(cK)
    pl.kernel(
        _sc_update,
        out_type=(),
        mesh=mesh,
        compiler_params=cp,
        scratch_types=[
            pltpu.VMEM((WREG,), jnp.int32),
            pltpu.VMEM((WREG,), jnp.int32),
            vec16f, vec16i,
            rows, rows, rows, rows,
            rows, rows, rows, rows,
            vec16f, vec16f, vec16f, vec16f,
            vec16f, vec16f, vec16f, vec16f,
            dma, dma, dma, dma, dma, dma, dma, dma,
        ],
    )(x, muK, cK, idx_b, lbl_b, cnt_b, mu_out, ck_out)
    return mu_out[...], ck_out[...]


# speedup vs baseline: 1.0793x; 1.0071x over previous
import jax
import jax.numpy as jnp
from jax import lax
from jax.experimental import pallas as pl
from jax.experimental.pallas import tpu as pltpu
from jax.experimental.pallas import tpu_sc as plsc

B, D, C = 16384, 512, 100000
L = 16
NW = 32
CPW = C // NW
CPAD = ((CPW + L) + L - 1) // L * L
NCHUNK = B // L
DCH = D // L
NBUF = 4


def _sc_update(x_hbm, y_hbm, mu_hbm, ck_hbm, mu_out, ck_out,
               y_v, sel_v, wsel_v, seen_v, lbl_s, idx_s, r_s,
               lblv0, lblv1, lblv2, lblv3, mur0, mur1, mur2, mur3,
               xr0, xr1, xr2, xr3, ckg0, ckg1, ckg2, ckg3,
               ckn0, ckn1, ckn2, ckn3,
               gsem0, gsem1, gsem2, gsem3, ssem0, ssem1, ssem2, ssem3):
    buf = (
        (lblv0, mur0, xr0, ckg0, ckn0, gsem0, ssem0),
        (lblv1, mur1, xr1, ckg1, ckn1, gsem1, ssem1),
        (lblv2, mur2, xr2, ckg2, ckn2, gsem2, ssem2),
        (lblv3, mur3, xr3, ckg3, ckn3, gsem3, ssem3),
    )
    wid = lax.axis_index("s") * 2 + lax.axis_index("c")
    lo = wid * CPW
    hi = lo + CPW

    pltpu.sync_copy(y_hbm, y_v)

    lanes = lax.iota(jnp.int32, L)
    zeros = lanes * 0

    def zero_step(i, c2):
        seen_v[pl.ds(i * L, L)] = zeros
        return c2

    lax.fori_loop(0, CPAD // L, zero_step, jnp.int32(0))

    def sel_step(c, cnt):
        yv = y_v[pl.ds(c * L, L)]
        m = ((yv >= lo) & (yv < hi)).astype(jnp.int32)
        pos = jnp.where(m > 0, cnt + jnp.cumsum(m) - 1, B + L)
        plsc.store_scatter(sel_v, [pos], lanes + c * L)
        return cnt + jnp.sum(m)

    cnt = lax.fori_loop(0, NCHUNK, sel_step, jnp.int32(0))

    last = jnp.maximum(cnt - 1, 0)
    pad = plsc.load_gather(sel_v, [zeros + last])
    sel_v[pl.ds(cnt, L)] = pad
    ngroups = (cnt + (L - 1)) >> 4

    def win_step(gi, wcnt):
        g = ngroups - 1 - gi
        idx = sel_v[pl.ds(g * L, L)]
        lbl = plsc.load_gather(y_v, [idx])
        rel = lbl - lo
        lbl_s[...] = lbl
        seen = plsc.load_gather(seen_v, [rel])
        dup_later = lbl != lbl
        for s in range(1, L):
            perm = jnp.minimum(lanes + s, L - 1)
            rl = plsc.load_gather(lbl_s, [perm])
            dup_later = dup_later | ((rl == lbl) & (lanes < (L - s)))
        win = (~dup_later) & (seen == 0)
        plsc.store_scatter(seen_v, [rel], zeros + 1)
        wm = win.astype(jnp.int32)
        pos = jnp.where(win, wcnt + jnp.cumsum(wm) - 1, B + L)
        plsc.store_scatter(wsel_v, [pos], idx)
        return wcnt + jnp.sum(wm)

    wcnt = lax.fori_loop(0, ngroups, win_step, jnp.int32(0))
    wlast = jnp.maximum(wcnt - 1, 0)
    wpad = plsc.load_gather(wsel_v, [zeros + wlast])
    wsel_v[pl.ds(wcnt, L)] = wpad
    wgroups = (wcnt + (L - 1)) >> 4

    def issue_gathers(g, k):
        lblv, mur, xr, ckg, _, gsem, _ = buf[k]
        idx = wsel_v[pl.ds(g * L, L)]
        lbl = plsc.load_gather(y_v, [idx])
        lblv[...] = lbl
        pltpu.async_copy(mu_hbm.at[lbl], mur, gsem)
        pltpu.async_copy(x_hbm.at[idx], xr, gsem)
        pltpu.async_copy(ck_hbm.at[lbl], ckg, gsem)

    def process(g, k):
        lblv, mur, xr, ckg, ckn, gsem, _ = buf[k]
        nk = (k + 1) % NBUF
        lbl = lblv[...]
        pltpu.make_async_copy(mu_hbm.at[lbl], mur, gsem).wait()
        pltpu.make_async_copy(x_hbm.at[lbl], xr, gsem).wait()
        pltpu.make_async_copy(ck_hbm.at[lbl], ckg, gsem).wait()

        @pl.when(g + 1 < wgroups)
        def _():
            @pl.when(g >= NBUF - 1)
            def _():
                plblv, pmur, _, _, pckn, _, pssem = buf[nk]
                lp = plblv[...]
                pltpu.make_async_copy(pmur, mu_out.at[lp], pssem).wait()
                pltpu.make_async_copy(pckn, ck_out.at[lp], pssem).wait()

            issue_gathers(g + 1, nk)

        ck1 = ckg[...] + 1.0
        r_s[...] = 1.0 / ck1
        ckn[...] = ck1

        def row_step(j, c2):
            rj = plsc.load_gather(r_s, [zeros + j])
            for cpos in range(DCH):
                mu = mur[j, pl.ds(cpos * L, L)]
                xx = xr[j, pl.ds(cpos * L, L)]
                mur[j, pl.ds(cpos * L, L)] = mu + (xx - mu) * rj
            return c2

        lax.fori_loop(0, L, row_step, jnp.int32(0))

        ssem = buf[k][6]
        pltpu.async_copy(mur, mu_out.at[lbl], ssem)
        pltpu.async_copy(ckn, ck_out.at[lbl], ssem)

    @pl.when(wgroups > 0)
    def _():
        issue_gathers(0, 0)

    def quad_step(p, carry):
        for k in range(NBUF):
            g = p * NBUF + k

            @pl.when(g < wgroups)
            def _(g=g, k=k):
                process(g, k)

        return carry

    lax.fori_loop(0, (wgroups + (NBUF - 1)) // NBUF, quad_step, jnp.int32(0))

    for t in range(1, NBUF + 1):
        for k in range(NBUF):
            @pl.when((wgroups >= t) & ((wgroups - t) % NBUF == k))
            def _(k=k):
                lblv, mur, _, _, ckn, _, ssem = buf[k]
                lp = lblv[...]
                pltpu.make_async_copy(mur, mu_out.at[lp], ssem).wait()
                pltpu.make_async_copy(ckn, ck_out.at[lp], ssem).wait()


def kernel(x, y, muK, cK):
    mu_out = jax.new_ref(muK)
    ck_out = jax.new_ref(cK)
    mesh = plsc.VectorSubcoreMesh(core_axis_name="c", subcore_axis_name="s",
                                  num_cores=2, num_subcores=16)
    vec16i = pltpu.VMEM((L,), jnp.int32)
    vec16f = pltpu.VMEM((L,), jnp.float32)
    rows = pltpu.VMEM((L, D), jnp.float32)
    dma = pltpu.SemaphoreType.DMA
    pl.kernel(
        _sc_update,
        out_type=(),
        mesh=mesh,
        compiler_params=pltpu.CompilerParams(needs_layout_passes=False),
        scratch_types=[
            pltpu.VMEM((B,), jnp.int32),
            pltpu.VMEM((B + 2 * L,), jnp.int32),
            pltpu.VMEM((B + 2 * L,), jnp.int32),
            pltpu.VMEM((CPAD,), jnp.int32),
            vec16i, vec16i, vec16f,
            vec16i, vec16i, vec16i, vec16i,
            rows, rows, rows, rows,
            rows, rows, rows, rows,
            vec16f, vec16f, vec16f, vec16f,
            vec16f, vec16f, vec16f, vec16f,
            dma, dma, dma, dma, dma, dma, dma, dma,
        ],
    )(x, y, muK, cK, mu_out, ck_out)
    return mu_out[...], ck_out[...]
